# baseline (speedup 1.0000x reference)
import jax
import jax.numpy as jnp
from jax import lax
from jax.experimental import pallas as pl
from jax.experimental.pallas import tpu as pltpu


def _body(x_ref, oidx_ref, pidx_ref, pn_ref, w_ref, out_ref, xb_ref):
    T, R = x_ref.shape
    NP = pn_ref.shape[0]
    NO = xb_ref.shape[1] // R
    j = pl.program_id(0)
    i = pl.program_id(1)

    @pl.when(j == 0)
    def _build_xb():
        xt = x_ref[...]
        pn = pn_ref[...]
        oh1 = (
            pidx_ref[:, 0:1] == lax.broadcasted_iota(jnp.int32, (T, NP), 1)
        ).astype(jnp.float32)
        oh2 = (
            pidx_ref[:, 1:2] == lax.broadcasted_iota(jnp.int32, (T, NP), 1)
        ).astype(jnp.float32)
        dmat = jnp.dot(xt, pn.T, preferred_element_type=jnp.float32)
        gmat = jnp.dot(pn, pn.T, preferred_element_type=jnp.float32)
        nvec = jnp.sum(
            gmat
            * (
                lax.broadcasted_iota(jnp.int32, (NP, NP), 0)
                == lax.broadcasted_iota(jnp.int32, (NP, NP), 1)
            ).astype(jnp.float32),
            axis=1,
            keepdims=True,
        )
        d1 = jnp.sum(oh1 * dmat, axis=1, keepdims=True)
        d2 = jnp.sum(oh2 * dmat, axis=1, keepdims=True)
        emat = jnp.dot(oh2, gmat.T, preferred_element_type=jnp.float32)
        d12 = jnp.sum(oh1 * emat, axis=1, keepdims=True)
        n1 = jnp.dot(oh1, nvec, preferred_element_type=jnp.float32) + 1e-8
        n2 = jnp.dot(oh2, nvec, preferred_element_type=jnp.float32) + 1e-8
        a = 2.0 * d1 / n1
        b = 2.0 * (d2 - a * d12) / n2
        xt = xt - jnp.dot(
            a * oh1 + b * oh2, pn, preferred_element_type=jnp.float32
        )
        ohe = (
            oidx_ref[...] == lax.broadcasted_iota(jnp.int32, (T, NO), 1)
        ).astype(jnp.float32)
        xb_ref[pl.ds(i * T, T), :] = jnp.concatenate(
            [xt * ohe[:, e : e + 1] for e in range(NO)], axis=1
        )

    out_ref[...] = jnp.dot(
        xb_ref[pl.ds(i * T, T), :], w_ref[...],
        preferred_element_type=jnp.float32,
    )


def kernel(x, output_idx, process_indices, process_neurons, output_neurons):
    B, S, R = x.shape
    NO, _, D = output_neurons.shape
    NP = process_neurons.shape[0]
    K = process_indices.shape[-1]
    xs = x.reshape(S, R)
    oidx = output_idx.reshape(S, 1)
    pidx = process_indices.reshape(S, K)
    wflat = output_neurons.reshape(NO * R, D)
    T = 512
    DT = 512
    grid = (D // DT, S // T)
    out = pl.pallas_call(
        _body,
        grid=grid,
        in_specs=[
            pl.BlockSpec((T, R), lambda j, i: (i, 0)),
            pl.BlockSpec((T, 1), lambda j, i: (i, 0)),
            pl.BlockSpec((T, K), lambda j, i: (i, 0)),
            pl.BlockSpec((NP, R), lambda j, i: (0, 0)),
            pl.BlockSpec((NO * R, DT), lambda j, i: (0, j)),
        ],
        out_specs=pl.BlockSpec((T, DT), lambda j, i: (i, j)),
        out_shape=jax.ShapeDtypeStruct((S, D), jnp.float32),
        scratch_shapes=[pltpu.VMEM((S, NO * R), jnp.float32)],
    )(xs, oidx, pidx, process_neurons, wflat)
    return out.reshape(B, S, D)
